# Initial kernel scaffold; baseline (speedup 1.0000x reference)
import jax
import jax.numpy as jnp
from jax.experimental import pallas as pl


def kernel(x, edge_index, batch, W1, b1, W2, b2, Wf1, bf1, Wf2, bf2):
    raise NotImplementedError("write your pallas kernel here")


# speedup vs baseline: 13.5274x; 13.5274x over previous
import functools

import jax
import jax.numpy as jnp
from jax import lax
from jax.experimental import pallas as pl
from jax.experimental.pallas import tpu as pltpu
from jax.experimental.pallas import tpu_sc as plsc

NC = 2
NS = 16
NW = NC * NS
LANES = 16
CH = 128

G = 64

def _mesh():
  return plsc.VectorSubcoreMesh(
      core_axis_name="c", subcore_axis_name="s", num_cores=NC, num_subcores=NS)


def _cdiv(a, b):
  return (a + b - 1) // b


def _make_deg_kernel(NR, C1):
  TR = NR // NS

  @functools.partial(
      pl.kernel,
      out_type=jax.ShapeDtypeStruct((NR,), jnp.float32),
      mesh=_mesh(),
      scratch_types=[
          pltpu.VMEM((C1, CH), jnp.int32),
          pltpu.VMEM((CH,), jnp.float32),
          pltpu.VMEM((TR,), jnp.float32),
          pltpu.VMEM_SHARED((NR,), jnp.float32),
      ],
  )
  def deg_kernel(dst_hbm, out_hbm, dst_v, ones_v, init_v, deg_sh):
    cid = lax.axis_index("c")
    sid = lax.axis_index("s")

    @pl.when(cid == 0)
    def _():
      for i in range(CH // LANES):
        ones_v[pl.ds(i * LANES, LANES)] = jnp.ones((LANES,), jnp.float32)
      for i in range(TR // LANES):
        init_v[pl.ds(i * LANES, LANES)] = jnp.ones((LANES,), jnp.float32)
      pltpu.sync_copy(init_v, deg_sh.at[pl.ds(sid * TR, TR)])
      pltpu.sync_copy(dst_hbm.at[sid], dst_v)
      plsc.subcore_barrier()

      def body(j, carry):
        pltpu.sync_copy(ones_v, deg_sh.at[dst_v.at[j]], add=True)
        return carry

      lax.fori_loop(0, C1, body, 0)
      plsc.subcore_barrier()
      pltpu.sync_copy(deg_sh.at[pl.ds(sid * TR, TR)],
                      out_hbm.at[pl.ds(sid * TR, TR)])

  return deg_kernel


def _make_agg_kernel(N, D, NR, C):
  TR = NR // NS
  ZR = 16

  @functools.partial(
      pl.kernel,
      out_type=jax.ShapeDtypeStruct((NC, NR, D), jnp.float32),
      mesh=_mesh(),
      scratch_types=[
          pltpu.VMEM((C, CH), jnp.int32),
          pltpu.VMEM((C, CH), jnp.int32),
          pltpu.VMEM((CH, D), jnp.float32),
          pltpu.VMEM((ZR, D), jnp.float32),
          pltpu.VMEM_SHARED((NR, D), jnp.float32),
      ],
  )
  def agg_kernel(hs_hbm, src_hbm, dst_hbm, out_hbm,
                 src_v, dst_v, rows_v, zero_v, acc_sh):
    cid = lax.axis_index("c")
    sid = lax.axis_index("s")
    wid = sid * NC + cid

    for i in range(ZR):
      for l in range(D // LANES):
        zero_v[i, pl.ds(l * LANES, LANES)] = jnp.zeros((LANES,), jnp.float32)

    def zinit(k, carry):
      pltpu.sync_copy(zero_v, acc_sh.at[pl.ds(sid * TR + k * ZR, ZR)])
      return carry

    lax.fori_loop(0, TR // ZR, zinit, 0)
    pltpu.sync_copy(src_hbm.at[wid], src_v)
    pltpu.sync_copy(dst_hbm.at[wid], dst_v)
    plsc.subcore_barrier()

    def body(j, carry):
      pltpu.sync_copy(hs_hbm.at[src_v.at[j]], rows_v)
      pltpu.sync_copy(rows_v, acc_sh.at[dst_v.at[j]], add=True)
      return carry

    lax.fori_loop(0, C, body, 0)
    plsc.subcore_barrier()
    pltpu.sync_copy(acc_sh.at[pl.ds(sid * TR, TR)],
                    out_hbm.at[cid, pl.ds(sid * TR, TR)])

  return agg_kernel


def _tc1_body(x_ref, w1_ref, deg_ref, hs_ref):
  dinv = lax.rsqrt(jnp.maximum(deg_ref[...], 1.0))
  hs_ref[...] = jnp.dot(x_ref[...], w1_ref[...],
                        preferred_element_type=jnp.float32) * dinv


def _tc2_body(N, parts_ref, hs1_ref, deg_ref, b1_ref, w2_ref, hs2_ref):
  dinv = lax.rsqrt(jnp.maximum(deg_ref[...], 1.0))
  agg = parts_ref[0, :N] + parts_ref[1, :N] + hs1_ref[...]
  h = jnp.maximum(agg * dinv + b1_ref[...], 0.0)
  hs2_ref[...] = jnp.dot(h, w2_ref[...],
                         preferred_element_type=jnp.float32) * dinv


def _tc3_body(N, parts_ref, hs2_ref, deg_ref, b2_ref, batch_ref,
              wf1_ref, bf1_ref, wf2_ref, bf2_ref, out_ref):
  dinv = lax.rsqrt(jnp.maximum(deg_ref[...], 1.0))
  h = jnp.maximum(
      (parts_ref[0, :N] + parts_ref[1, :N] + hs2_ref[...]) * dinv
      + b2_ref[...], 0.0)
  gi = lax.broadcasted_iota(jnp.int32, (G, N), 0)
  onehot = (gi == batch_ref[...]).astype(jnp.float32)
  seg = jnp.dot(onehot, h, preferred_element_type=jnp.float32)
  cnt = jnp.sum(onehot, axis=1, keepdims=True)
  p = seg / jnp.maximum(cnt, 1.0)
  o1 = jnp.maximum(
      jnp.dot(p, wf1_ref[...], preferred_element_type=jnp.float32)
      + bf1_ref[...], 0.0)
  out_ref[...] = (jnp.dot(o1, wf2_ref[...], preferred_element_type=jnp.float32)
                  + bf2_ref[...])


def kernel(x, edge_index, batch, W1, b1, W2, b2, Wf1, bf1, Wf2, bf2):
  N, D = x.shape
  H1 = W1.shape[1]
  H2 = Wf1.shape[1]
  E = edge_index.shape[1]

  x = x.astype(jnp.float32)
  src = edge_index[0].astype(jnp.int32)
  dst = edge_index[1].astype(jnp.int32)
  batch2 = batch.astype(jnp.int32).reshape(1, N)

  NR = _cdiv(N + 1, NS * LANES) * NS * LANES

  C = _cdiv(_cdiv(E, NW), CH)
  EP = NW * C * CH
  src_a = jnp.concatenate(
      [src, jnp.zeros((EP - E,), jnp.int32)]).reshape(NW, C, CH)
  dst_a = jnp.concatenate(
      [dst, jnp.full((EP - E,), N, jnp.int32)]).reshape(NW, C, CH)

  C1 = _cdiv(_cdiv(E, NS), CH)
  EP1 = NS * C1 * CH
  dst_d = jnp.concatenate(
      [dst, jnp.full((EP1 - E,), N, jnp.int32)]).reshape(NS, C1, CH)

  deg_full = _make_deg_kernel(NR, C1)(dst_d)
  degn = deg_full[:N].reshape(N, 1)

  tc1 = pl.pallas_call(
      _tc1_body,
      out_shape=jax.ShapeDtypeStruct((N, H1), jnp.float32))
  hs1 = tc1(x, W1, degn)

  agg = _make_agg_kernel(N, H1, NR, C)
  parts1 = agg(hs1, src_a, dst_a)

  tc2 = pl.pallas_call(
      functools.partial(_tc2_body, N),
      out_shape=jax.ShapeDtypeStruct((N, H1), jnp.float32))
  hs2 = tc2(parts1, hs1, degn, b1.reshape(1, H1), W2)

  parts2 = agg(hs2, src_a, dst_a)

  tc3 = pl.pallas_call(
      functools.partial(_tc3_body, N),
      out_shape=jax.ShapeDtypeStruct((G, 1), jnp.float32))
  out = tc3(parts2, hs2, degn, b2.reshape(1, H1), batch2,
            Wf1, bf1.reshape(1, H2), Wf2, bf2.reshape(1, 1))
  return out
